# Initial kernel scaffold; baseline (speedup 1.0000x reference)
import jax
import jax.numpy as jnp
from jax.experimental import pallas as pl


def kernel(h, coords, edge_index, edge_type, edge_bond_type, edge_bond_conjugated, edge_bond_in_ring, edge_bond_stereo, edge_ref_dist, t_emb, E_type, E_bt, E_conj, E_ring, E_st, ref_W, ref_b, mlp_W1, mlp_b1, mlp_W2, mlp_b2, M_sv, M_vs, M_sl, P0, P1o, P1e, P2, A_W, A_b):
    raise NotImplementedError("write your pallas kernel here")


# speedup vs baseline: 2.5339x; 2.5339x over previous
import functools

import jax
import jax.numpy as jnp
import numpy as np
from jax import lax
from jax.experimental import pallas as pl
from jax.experimental.pallas import tpu as pltpu
from jax.experimental.pallas import tpu_sc as plsc

S, V, L = 128, 16, 8
TEMB = 128
NRBF = 16
D = S + 6 * V + 5 * L
N_NODES = 10000
N_EDGES = 160000

NPAD = 10240
EPAD = 163840
CH = 128
NW_SC = 32
DUMP = NPAD
SPROWS = 10256
HALF = 144
WS = 288
WD = 160
BE = 512
BN = 256
NBUF_G = 3
NBUF_S = 2

_SQRT3 = float(np.sqrt(3.0))
_C15 = float(np.sqrt(15.0))
_C5H = float(np.sqrt(5.0) / 2.0)

def _make_perm():
    p = list(range(S))
    for k in range(3):
        for v in range(V):
            p.append(S + 3 * v + k)
    for k in range(3):
        for v in range(V):
            p.append(S + 3 * V + 3 * v + k)
    for k in range(5):
        for l in range(L):
            p.append(S + 6 * V + 5 * l + k)
    return np.array(p, dtype=np.int32)

_PERM = _make_perm()
_INVPERM = np.argsort(_PERM).astype(np.int32)


def _k0_body(h0_ref, te_ref, co_ref, whd_ref, wte_ref, b1_ref, out_ref):
    pre = (jnp.dot(h0_ref[...], whd_ref[...], preferred_element_type=jnp.float32)
           + jnp.dot(te_ref[...], wte_ref[...], preferred_element_type=jnp.float32)
           + b1_ref[...])
    out_ref[:, 0:128] = pre.astype(jnp.bfloat16)
    out_ref[:, 128:136] = co_ref[...].astype(jnp.bfloat16)
    out_ref[:, 136:WD] = jnp.zeros((BN, WD - 136), jnp.bfloat16)


def _node_pre(h0, te, co8, w_hd, w_te, b1f):
    grid = NPAD // BN
    return pl.pallas_call(
        _k0_body,
        grid=(grid,),
        in_specs=[
            pl.BlockSpec((BN, 128), lambda i: (i, 0)),
            pl.BlockSpec((BN, 128), lambda i: (i, 0)),
            pl.BlockSpec((BN, 8), lambda i: (i, 0)),
            pl.BlockSpec((128, 128), lambda i: (0, 0)),
            pl.BlockSpec((128, 128), lambda i: (0, 0)),
            pl.BlockSpec((1, 128), lambda i: (0, 0)),
        ],
        out_specs=pl.BlockSpec((BN, WD), lambda i: (i, 0)),
        out_shape=jax.ShapeDtypeStruct((NPAD, WD), jnp.bfloat16),
        interpret=False,
    )(h0, te, co8, w_hd, w_te, b1f)


def _sc_gather(ts, td, srcp, dstp):
    mesh = plsc.VectorSubcoreMesh(core_axis_name="c", subcore_axis_name="s",
                                  num_cores=2, num_subcores=16)
    per_w = EPAD // NW_SC
    n_ch = per_w // CH

    @functools.partial(
        pl.kernel,
        out_type=(jax.ShapeDtypeStruct((EPAD, WS), jnp.bfloat16),
                  jax.ShapeDtypeStruct((EPAD, WD), jnp.bfloat16)),
        mesh=mesh,
        compiler_params=pltpu.CompilerParams(use_tc_tiling_on_sc=False),
        scratch_types=(
            [pltpu.VMEM((n_ch, CH), jnp.int32),
             pltpu.VMEM((n_ch, CH), jnp.int32)]
            + [pltpu.VMEM((CH, WS), jnp.bfloat16) for _ in range(NBUF_G)]
            + [pltpu.VMEM((CH, WD), jnp.bfloat16) for _ in range(NBUF_G)]
            + [pltpu.SemaphoreType.DMA for _ in range(2 * NBUF_G)]
        ),
    )
    def k(ts_hbm, td_hbm, src_hbm, dst_hbm, gs_hbm, gd_hbm,
          idxs_v, idxd_v, *bufs):
        rows_s = bufs[0:NBUF_G]
        rows_d = bufs[NBUF_G:2 * NBUF_G]
        gsem = bufs[2 * NBUF_G:3 * NBUF_G]
        wsem = bufs[3 * NBUF_G:4 * NBUF_G]
        wid = lax.axis_index("s") * 2 + lax.axis_index("c")
        base = wid * per_w

        pltpu.sync_copy(src_hbm.at[wid], idxs_v)
        pltpu.sync_copy(dst_hbm.at[wid], idxd_v)

        def start_gather(ch, b):
            pltpu.async_copy(ts_hbm.at[idxs_v.at[ch]], rows_s[b], gsem[b])
            pltpu.async_copy(td_hbm.at[idxd_v.at[ch]], rows_d[b], gsem[b])

        def wait_gather(b):
            pltpu.make_async_copy(ts_hbm.at[idxs_v.at[0]], rows_s[b], gsem[b]).wait()
            pltpu.make_async_copy(td_hbm.at[idxd_v.at[0]], rows_d[b], gsem[b]).wait()

        def start_write(ch, b):
            off = base + ch * CH
            pltpu.async_copy(rows_s[b], gs_hbm.at[pl.ds(off, CH)], wsem[b])
            pltpu.async_copy(rows_d[b], gd_hbm.at[pl.ds(off, CH)], wsem[b])

        def wait_write(b):
            pltpu.make_async_copy(rows_s[b], gs_hbm.at[pl.ds(0, CH)], wsem[b]).wait()
            pltpu.make_async_copy(rows_d[b], gd_hbm.at[pl.ds(0, CH)], wsem[b]).wait()

        for b in range(NBUF_G):
            start_gather(b, b)

        n_out = (n_ch + NBUF_G - 1) // NBUF_G

        def body(j, carry):
            for b in range(NBUF_G):
                ch = j * NBUF_G + b

                @pl.when(ch < n_ch)
                def _():
                    wait_gather(b)
                    start_write(ch, b)
                    wait_write(b)

                    @pl.when(ch + NBUF_G < n_ch)
                    def _():
                        start_gather(ch + NBUF_G, b)

            return carry

        lax.fori_loop(0, n_out, body, 0)

    return k(ts, td, srcp, dstp)


def _k2_body(gs_ref, gd_ref, at_ref, vals_ref, sel_ref, sw32_ref, swref_ref,
             swrbf_ref, whs_ref, w2b_ref, b2b_ref, msvle_ref, r1e_ref,
             r2e_ref, tuf1_ref, tuf2_ref, tm7_ref, tsum_ref, mvs_ref,
             outa_ref, outb_ref):
    f32 = jnp.float32
    Gs = gs_ref[...].astype(f32)
    Gd = gd_ref[...].astype(f32)
    at = at_ref[...]
    h0s = Gs[:, 0:128]
    cs = Gs[:, 264:267]
    cd = Gd[:, 128:131]
    diff = cd - cs
    r2 = jnp.sum(diff * diff, axis=1, keepdims=True)
    dist = jnp.sqrt(r2)
    u = diff / (dist + 1e-9)

    centers = lax.broadcasted_iota(jnp.int32, (BE, NRBF), 1).astype(f32) * (5.0 / (NRBF - 1))
    width = 5.0 / NRBF
    rbf = jnp.exp(-(((dist - centers) / width) ** 2))

    spread = jnp.dot(at, sel_ref[...], preferred_element_type=f32)
    oh32 = (spread == vals_ref[...]).astype(f32)
    refd = at[:, 5:6]
    dd = dist - refd
    has = (refd > 0).astype(f32)
    eref3 = jnp.concatenate([jnp.abs(dd), dd, has], axis=1)

    hid_in = (jnp.dot(oh32, sw32_ref[...], preferred_element_type=f32)
              + jnp.dot(eref3, swref_ref[...], preferred_element_type=f32)
              + jnp.dot(rbf, swrbf_ref[...], preferred_element_type=f32)
              + jnp.dot(h0s, whs_ref[...], preferred_element_type=f32)
              + Gd[:, 0:128])
    hid = hid_in * (1.0 / (1.0 + jnp.exp(-hid_in)))
    w = jnp.dot(hid, w2b_ref[...], preferred_element_type=f32) + b2b_ref[...]

    H1O = Gs[:, 128:176]
    HL = Gs[:, 128:264]
    urot = jnp.concatenate([u[:, 1:3], u[:, 0:1]], axis=1)
    m7 = jnp.concatenate([u * u, u * urot, jnp.ones((BE, 1), f32)], axis=1)
    E1 = (jnp.dot(h0s, msvle_ref[...], preferred_element_type=f32)
          + jnp.dot(H1O, r1e_ref[...], preferred_element_type=f32))
    E2 = jnp.dot(H1O, r2e_ref[...], preferred_element_type=f32)
    F1 = (jnp.dot(u, tuf1_ref[...], preferred_element_type=f32)
          + jnp.dot(m7, tm7_ref[...], preferred_element_type=f32))
    F2 = jnp.dot(u, tuf2_ref[...], preferred_element_type=f32)
    Q = E1 * F1 - E2 * F2
    A1 = w[:, 128:264]
    A2 = w[:, 264:400]
    msgL = A1 * HL + A2 * Q

    D = H1O * F1[:, 0:48]
    dot1 = jnp.dot(D, tsum_ref[...], preferred_element_type=f32)
    wc = w[:, 400:416]
    msg0 = w[:, 0:128] * h0s + jnp.dot(wc * dot1, mvs_ref[...],
                                       preferred_element_type=f32)

    q = 0.25
    outa_ref[:, 0:128] = q * msg0
    outa_ref[:, 128:144] = jnp.zeros((BE, 16), f32)
    outb_ref[:, 0:136] = q * msgL
    outb_ref[:, 136:144] = jnp.zeros((BE, 8), f32)


def _edge_compute(gs, gd, attr, vals, sel, sw32, swref, swrbf, w_hs,
                  w2b, b2b, msvle, r1e, r2e, tuf1, tuf2, tm7, tsum, m_vs):
    grid = EPAD // BE
    const = lambda shape: pl.BlockSpec(shape, lambda i: (0,) * len(shape))
    return pl.pallas_call(
        _k2_body,
        grid=(grid,),
        in_specs=[
            pl.BlockSpec((BE, WS), lambda i: (i, 0)),
            pl.BlockSpec((BE, WD), lambda i: (i, 0)),
            pl.BlockSpec((BE, 8), lambda i: (i, 0)),
            const((1, 32)), const((8, 32)), const((32, 128)),
            const((3, 128)), const((16, 128)), const((128, 128)),
            const((128, 416)), const((1, 416)), const((128, 136)),
            const((48, 136)), const((48, 136)), const((3, 136)),
            const((3, 136)), const((7, 136)), const((48, 16)),
            const((16, 128)),
        ],
        out_specs=[
            pl.BlockSpec((BE, HALF), lambda i: (i, 0)),
            pl.BlockSpec((BE, HALF), lambda i: (i, 0)),
        ],
        out_shape=[
            jax.ShapeDtypeStruct((EPAD, HALF), jnp.float32),
            jax.ShapeDtypeStruct((EPAD, HALF), jnp.float32),
        ],
        interpret=False,
    )(gs, gd, attr, vals, sel, sw32, swref, swrbf, w_hs, w2b, b2b,
      msvle, r1e, r2e, tuf1, tuf2, tm7, tsum, m_vs)


def _sc_scatter(msga, msgb, dstp, zrows):
    mesh = plsc.VectorSubcoreMesh(core_axis_name="c", subcore_axis_name="s",
                                  num_cores=2, num_subcores=16)
    per_t = EPAD // 16
    n_ch = per_t // CH
    zch = SPROWS // 16
    out_ch = NPAD // CH // 16

    iblk = 8

    @functools.partial(
        pl.kernel,
        out_type=(jax.ShapeDtypeStruct((NPAD, HALF), jnp.float32),
                  jax.ShapeDtypeStruct((NPAD, HALF), jnp.float32)),
        mesh=mesh,
        compiler_params=pltpu.CompilerParams(use_tc_tiling_on_sc=False),
        scratch_types=(
            [pltpu.VMEM((iblk, CH), jnp.int32)]
            + [pltpu.VMEM((CH, HALF), jnp.float32) for _ in range(NBUF_S)]
            + [pltpu.VMEM_SHARED((SPROWS, HALF), jnp.float32)]
            + [pltpu.SemaphoreType.DMA for _ in range(NBUF_S)]
        ),
    )
    def k(ma_hbm, mb_hbm, dst_hbm, z_hbm, outa_hbm, outb_hbm,
          idx_v, *bufs):
        rows = bufs[0:NBUF_S]
        acc_sh = bufs[NBUF_S]
        rsem = bufs[NBUF_S + 1:2 * NBUF_S + 1]
        c = lax.axis_index("c")
        t = lax.axis_index("s")
        pltpu.sync_copy(z_hbm, acc_sh.at[pl.ds(t * zch, zch)])
        plsc.subcore_barrier()

        base = t * per_t

        def start_read(ch, b):
            off = base + ch * CH

            @pl.when(c == 0)
            def _():
                pltpu.async_copy(ma_hbm.at[pl.ds(off, CH)], rows[b], rsem[b])

            @pl.when(c == 1)
            def _():
                pltpu.async_copy(mb_hbm.at[pl.ds(off, CH)], rows[b], rsem[b])

        def wait_read(b):
            pltpu.make_async_copy(ma_hbm.at[pl.ds(0, CH)], rows[b], rsem[b]).wait()

        pltpu.sync_copy(dst_hbm.at[t, pl.ds(0, iblk)], idx_v)
        for b in range(NBUF_S):
            start_read(b, b)

        n_it = n_ch // NBUF_S
        per_blk = iblk // NBUF_S

        def body(j, carry):
            for b in range(NBUF_S):
                ch = j * NBUF_S + b
                row = (j % per_blk) * NBUF_S + b
                wait_read(b)
                pltpu.sync_copy(rows[b], acc_sh.at[idx_v.at[row]], add=True)

                @pl.when(ch + NBUF_S < n_ch)
                def _():
                    start_read(ch + NBUF_S, b)

            nxt = (j + 1) * NBUF_S

            @pl.when((nxt % iblk == 0) & (nxt < n_ch))
            def _():
                pltpu.sync_copy(dst_hbm.at[t, pl.ds(nxt, iblk)], idx_v)

            return carry

        lax.fori_loop(0, n_it, body, 0)
        plsc.subcore_barrier()

        def wbody(i, carry):
            cidx = t + i * 16

            @pl.when(c == 0)
            def _():
                pltpu.sync_copy(acc_sh.at[pl.ds(cidx * CH, CH)],
                                outa_hbm.at[pl.ds(cidx * CH, CH)])

            @pl.when(c == 1)
            def _():
                pltpu.sync_copy(acc_sh.at[pl.ds(cidx * CH, CH)],
                                outb_hbm.at[pl.ds(cidx * CH, CH)])

            return carry

        lax.fori_loop(0, out_ch, wbody, 0)

    return k(msga, msgb, dstp, zrows)


def _k4_body(aa_ref, ab_ref, hk_ref, te_ref, pa_ref, pb_ref, aw_ref, ab2_ref,
             out_ref):
    A = aa_ref[...]
    Bm = ab_ref[...]
    H = hk_ref[...]
    T = te_ref[...]
    U = (jnp.dot(A, pa_ref[...], preferred_element_type=jnp.float32)
         + jnp.dot(Bm, pb_ref[...], preferred_element_type=jnp.float32))
    u0 = U[:, 0:128]
    u0 = u0 * (1.0 / (1.0 + jnp.exp(-u0)))
    hn0 = H[:, 0:128] + u0
    hn1 = H[:, 128:264] + U[:, 128:264]

    p = jnp.dot(T, aw_ref[...], preferred_element_type=jnp.float32) + ab2_ref[...]
    scale = p[:, 0:128]
    shift = p[:, 128:256]
    g1o = p[:, 256:272]
    g1e = p[:, 272:288]
    g2 = p[:, 288:296]

    mu = jnp.mean(hn0, axis=1, keepdims=True)
    xc = hn0 - mu
    var = jnp.mean(xc * xc, axis=1, keepdims=True)
    o0 = xc * lax.rsqrt(var + 1e-5) * (1.0 + scale) + shift

    b1o = hn1[:, 0:48]
    b1e = hn1[:, 48:96]
    b2_ = hn1[:, 96:136]
    n1o = jnp.sqrt(jnp.sum(b1o * b1o, axis=1, keepdims=True) / 16.0 + 1e-5)
    n1e = jnp.sqrt(jnp.sum(b1e * b1e, axis=1, keepdims=True) / 16.0 + 1e-5)
    n2 = jnp.sqrt(jnp.sum(b2_ * b2_, axis=1, keepdims=True) / 8.0 + 1e-5)
    g1o3 = jnp.concatenate([1.0 + g1o] * 3, axis=1)
    g1e3 = jnp.concatenate([1.0 + g1e] * 3, axis=1)
    g25 = jnp.concatenate([1.0 + g2] * 5, axis=1)

    out_ref[:, 0:128] = o0
    out_ref[:, 128:176] = b1o / n1o * g1o3
    out_ref[:, 176:224] = b1e / n1e * g1e3
    out_ref[:, 224:264] = b2_ / n2 * g25


def _node_update(agga, aggb, hk, te, pa, pb, aw, ab2):
    grid = NPAD // BN
    return pl.pallas_call(
        _k4_body,
        grid=(grid,),
        in_specs=[
            pl.BlockSpec((BN, HALF), lambda i: (i, 0)),
            pl.BlockSpec((BN, HALF), lambda i: (i, 0)),
            pl.BlockSpec((BN, 264), lambda i: (i, 0)),
            pl.BlockSpec((BN, 128), lambda i: (i, 0)),
            pl.BlockSpec((HALF, 264), lambda i: (0, 0)),
            pl.BlockSpec((HALF, 264), lambda i: (0, 0)),
            pl.BlockSpec((128, 296), lambda i: (0, 0)),
            pl.BlockSpec((1, 296), lambda i: (0, 0)),
        ],
        out_specs=pl.BlockSpec((BN, 264), lambda i: (i, 0)),
        out_shape=jax.ShapeDtypeStruct((NPAD, 264), jnp.float32),
        interpret=False,
    )(agga, aggb, hk, te, pa, pb, aw, ab2)


def kernel(h, coords, edge_index, edge_type, edge_bond_type,
           edge_bond_conjugated, edge_bond_in_ring, edge_bond_stereo,
           edge_ref_dist, t_emb, E_type, E_bt, E_conj, E_ring, E_st,
           ref_W, ref_b, mlp_W1, mlp_b1, mlp_W2, mlp_b2,
           M_sv, M_vs, M_sl, P0, P1o, P1e, P2, A_W, A_b):
    f32 = jnp.float32
    bf16 = jnp.bfloat16
    n = h.shape[0]
    e = edge_index.shape[1]

    h_k = h[:, _PERM]
    h_kp = jnp.pad(h_k, ((0, NPAD - n), (0, 0)))
    co8 = jnp.pad(coords.astype(f32), ((0, NPAD - n), (0, 5)))
    te_p = jnp.pad(t_emb, ((0, NPAD - n), (0, 0)))
    tsrc = jnp.concatenate(
        [h_kp.astype(bf16), co8.astype(bf16),
         jnp.zeros((NPAD, WS - 272), bf16)], axis=1)

    src = edge_index[0].astype(jnp.int32)
    dst = edge_index[1].astype(jnp.int32)
    per_w = EPAD // NW_SC
    srcp = jnp.pad(src, (0, EPAD - e)).reshape(NW_SC, per_w // CH, CH)
    dst_g = jnp.pad(dst, (0, EPAD - e)).reshape(NW_SC, per_w // CH, CH)
    per_t = EPAD // 16
    dst_s = jnp.pad(dst, (0, EPAD - e),
                    constant_values=DUMP).reshape(16, per_t // CH, CH)

    attr = jnp.concatenate([
        edge_type[:, None].astype(f32),
        edge_bond_type[:, None].astype(f32),
        edge_bond_conjugated[:, None].astype(f32),
        edge_bond_in_ring[:, None].astype(f32),
        edge_bond_stereo[:, None].astype(f32),
        edge_ref_dist[:, None].astype(f32),
        jnp.zeros((e, 2), f32),
    ], axis=1)
    attr = jnp.pad(attr, ((0, EPAD - e), (0, 0)))

    w1_rbf = mlp_W1[0:16]
    w1_et = mlp_W1[16:32]
    w1_bt = mlp_W1[32:40]
    w1_cj = mlp_W1[40:44]
    w1_rg = mlp_W1[44:48]
    w1_st = mlp_W1[48:52]
    w1_rf = mlp_W1[52:60]
    w1_hs = mlp_W1[60:188]
    w1_hd = mlp_W1[188:316]
    w1_te = mlp_W1[316:444]
    sw32 = jnp.concatenate([
        E_type @ w1_et, E_bt @ w1_bt, E_conj @ w1_cj, E_ring @ w1_rg,
        E_st @ w1_st, jnp.zeros((6, 128), f32),
    ], axis=0)
    swref = ref_W @ w1_rf
    b1f = (mlp_b1 + ref_b @ w1_rf)[None, :]

    seln = np.zeros((8, 32), np.float32)
    valn = np.zeros((1, 32), np.float32)
    _offs = [(0, 0, 9), (1, 9, 6), (2, 15, 3), (3, 18, 3), (4, 21, 5)]
    for col, off, kk in _offs:
        for j in range(kk):
            seln[col, off + j] = 1.0
            valn[0, off + j] = float(j)
    for j in range(26, 32):
        seln[6, j] = 1.0
        valn[0, j] = -1.0
    sel = jnp.asarray(seln)
    vals = jnp.asarray(valn)

    expn = np.zeros((224, 416), np.float32)
    for j in range(128):
        expn[j, j] = 1.0
    for k in range(3):
        for v in range(16):
            expn[144 + v, 128 + 16 * k + v] = 1.0
            expn[176 + v, 176 + 16 * k + v] = 1.0
            expn[128 + v, 264 + 16 * k + v] = 1.0
            expn[192 + v, 312 + 16 * k + v] = 1.0
    for k in range(5):
        for l in range(8):
            expn[208 + l, 224 + 8 * k + l] = 1.0
            expn[216 + l, 360 + 8 * k + l] = 1.0
    for v in range(16):
        expn[160 + v, 400 + v] = 1.0
    expj = jnp.asarray(expn)
    w2b = mlp_W2 @ expj
    b2b = (mlp_b2 @ expj)[None, :]

    tile_sv = np.zeros((16, 48), np.float32)
    for k in range(3):
        for v in range(16):
            tile_sv[v, 16 * k + v] = 1.0
    tile_sl = np.zeros((8, 40), np.float32)
    for k in range(5):
        for l in range(8):
            tile_sl[l, 8 * k + l] = 1.0
    msvle = jnp.concatenate([
        M_sv @ jnp.asarray(tile_sv), jnp.zeros((128, 48), f32),
        M_sl @ jnp.asarray(tile_sl)], axis=1)

    r1n = np.zeros((48, 136), np.float32)
    r2n = np.zeros((48, 136), np.float32)
    for k in range(3):
        for v in range(16):
            r1n[16 * ((k + 1) % 3) + v, 48 + 16 * k + v] = 1.0
            r2n[16 * ((k + 2) % 3) + v, 48 + 16 * k + v] = 1.0
    r1e = jnp.asarray(r1n)
    r2e = jnp.asarray(r2n)

    t1n = np.zeros((3, 136), np.float32)
    t2n = np.zeros((3, 136), np.float32)
    for k in range(3):
        for v in range(16):
            t1n[k, 16 * k + v] = _SQRT3
            t1n[(k + 2) % 3, 48 + 16 * k + v] = _SQRT3
            t2n[(k + 1) % 3, 48 + 16 * k + v] = _SQRT3
    tuf1 = jnp.asarray(t1n)
    tuf2 = jnp.asarray(t2n)

    tmn = np.zeros((7, 136), np.float32)
    y2rows = [
        {3: _C15},
        {4: _C15},
        {2: 3.0 * _C5H, 6: -_C5H},
        {5: _C15},
        {0: _C15 / 2.0, 1: -_C15 / 2.0},
    ]
    for k in range(5):
        for row, coef in y2rows[k].items():
            for l in range(8):
                tmn[row, 96 + 8 * k + l] = coef
    tm7 = jnp.asarray(tmn)

    tsn = np.zeros((48, 16), np.float32)
    for k in range(3):
        for v in range(16):
            tsn[16 * k + v, v] = 1.0
    tsum = jnp.asarray(tsn)

    pbig = jax.scipy.linalg.block_diag(
        P0, P1o, P1o, P1o, P1e, P1e, P1e, P2, P2, P2, P2, P2)
    pa = jnp.pad(pbig[0:128], ((0, HALF - 128), (0, 0)))
    pb = jnp.pad(pbig[128:264], ((0, HALF - 136), (0, 0)))
    ab2 = A_b[None, :]
    zrows = jnp.zeros((SPROWS // 16, HALF), f32)

    tdst = _node_pre(h_kp[:, 0:128], te_p, co8, w1_hd, w1_te, b1f)
    gs, gd = _sc_gather(tsrc, tdst, srcp, dst_g)
    msga, msgb = _edge_compute(gs, gd, attr, vals, sel, sw32, swref, w1_rbf,
                               w1_hs, w2b, b2b, msvle, r1e, r2e, tuf1, tuf2,
                               tm7, tsum, M_vs)
    agga, aggb = _sc_scatter(msga, msgb, dst_s, zrows)
    out_k = _node_update(agga, aggb, h_kp, te_p, pa, pb, A_W, ab2)
    return out_k[:n][:, _INVPERM]


# speedup vs baseline: 2.5706x; 1.0145x over previous
import functools

import jax
import jax.numpy as jnp
import numpy as np
from jax import lax
from jax.experimental import pallas as pl
from jax.experimental.pallas import tpu as pltpu
from jax.experimental.pallas import tpu_sc as plsc

S, V, L = 128, 16, 8
TEMB = 128
NRBF = 16
D = S + 6 * V + 5 * L
N_NODES = 10000
N_EDGES = 160000

NPAD = 10240
EPAD = 163840
CH = 128
NW_SC = 32
DUMP = NPAD
SPROWS = 10256
HALF = 144
WS = 288
WD = 160
BE = 512
BN = 256
NBUF_G = 3
NBUF_S = 2

_SQRT3 = float(np.sqrt(3.0))
_C15 = float(np.sqrt(15.0))
_C5H = float(np.sqrt(5.0) / 2.0)

def _make_perm():
    p = list(range(S))
    for k in range(3):
        for v in range(V):
            p.append(S + 3 * v + k)
    for k in range(3):
        for v in range(V):
            p.append(S + 3 * V + 3 * v + k)
    for k in range(5):
        for l in range(L):
            p.append(S + 6 * V + 5 * l + k)
    return np.array(p, dtype=np.int32)

_PERM = _make_perm()
_INVPERM = np.argsort(_PERM).astype(np.int32)


def _k0_body(h0_ref, te_ref, co_ref, whd_ref, wte_ref, b1_ref, out_ref):
    bf16 = jnp.bfloat16
    pre = (jnp.dot(h0_ref[...].astype(bf16), whd_ref[...].astype(bf16),
                   preferred_element_type=jnp.float32)
           + jnp.dot(te_ref[...].astype(bf16), wte_ref[...].astype(bf16),
                     preferred_element_type=jnp.float32)
           + b1_ref[...])
    out_ref[:, 0:128] = pre.astype(jnp.bfloat16)
    out_ref[:, 128:136] = co_ref[...].astype(jnp.bfloat16)
    out_ref[:, 136:WD] = jnp.zeros((BN, WD - 136), jnp.bfloat16)


def _node_pre(h0, te, co8, w_hd, w_te, b1f):
    grid = NPAD // BN
    return pl.pallas_call(
        _k0_body,
        grid=(grid,),
        in_specs=[
            pl.BlockSpec((BN, 128), lambda i: (i, 0)),
            pl.BlockSpec((BN, 128), lambda i: (i, 0)),
            pl.BlockSpec((BN, 8), lambda i: (i, 0)),
            pl.BlockSpec((128, 128), lambda i: (0, 0)),
            pl.BlockSpec((128, 128), lambda i: (0, 0)),
            pl.BlockSpec((1, 128), lambda i: (0, 0)),
        ],
        out_specs=pl.BlockSpec((BN, WD), lambda i: (i, 0)),
        out_shape=jax.ShapeDtypeStruct((NPAD, WD), jnp.bfloat16),
        interpret=False,
    )(h0, te, co8, w_hd, w_te, b1f)


def _sc_gather(ts, td, srcp, dstp):
    mesh = plsc.VectorSubcoreMesh(core_axis_name="c", subcore_axis_name="s",
                                  num_cores=2, num_subcores=16)
    per_w = EPAD // NW_SC
    n_ch = per_w // CH

    @functools.partial(
        pl.kernel,
        out_type=(jax.ShapeDtypeStruct((EPAD, WS), jnp.bfloat16),
                  jax.ShapeDtypeStruct((EPAD, WD), jnp.bfloat16)),
        mesh=mesh,
        compiler_params=pltpu.CompilerParams(use_tc_tiling_on_sc=False),
        scratch_types=(
            [pltpu.VMEM((n_ch, CH), jnp.int32),
             pltpu.VMEM((n_ch, CH), jnp.int32)]
            + [pltpu.VMEM((CH, WS), jnp.bfloat16) for _ in range(NBUF_G)]
            + [pltpu.VMEM((CH, WD), jnp.bfloat16) for _ in range(NBUF_G)]
            + [pltpu.SemaphoreType.DMA for _ in range(2 * NBUF_G)]
        ),
    )
    def k(ts_hbm, td_hbm, src_hbm, dst_hbm, gs_hbm, gd_hbm,
          idxs_v, idxd_v, *bufs):
        rows_s = bufs[0:NBUF_G]
        rows_d = bufs[NBUF_G:2 * NBUF_G]
        gsem = bufs[2 * NBUF_G:3 * NBUF_G]
        wsem = bufs[3 * NBUF_G:4 * NBUF_G]
        wid = lax.axis_index("s") * 2 + lax.axis_index("c")
        base = wid * per_w

        pltpu.sync_copy(src_hbm.at[wid], idxs_v)
        pltpu.sync_copy(dst_hbm.at[wid], idxd_v)

        def start_gather(ch, b):
            pltpu.async_copy(ts_hbm.at[idxs_v.at[ch]], rows_s[b], gsem[b])
            pltpu.async_copy(td_hbm.at[idxd_v.at[ch]], rows_d[b], gsem[b])

        def wait_gather(b):
            pltpu.make_async_copy(ts_hbm.at[idxs_v.at[0]], rows_s[b], gsem[b]).wait()
            pltpu.make_async_copy(td_hbm.at[idxd_v.at[0]], rows_d[b], gsem[b]).wait()

        def start_write(ch, b):
            off = base + ch * CH
            pltpu.async_copy(rows_s[b], gs_hbm.at[pl.ds(off, CH)], wsem[b])
            pltpu.async_copy(rows_d[b], gd_hbm.at[pl.ds(off, CH)], wsem[b])

        def wait_write(b):
            pltpu.make_async_copy(rows_s[b], gs_hbm.at[pl.ds(0, CH)], wsem[b]).wait()
            pltpu.make_async_copy(rows_d[b], gd_hbm.at[pl.ds(0, CH)], wsem[b]).wait()

        for b in range(NBUF_G):
            start_gather(b, b)

        n_out = (n_ch + NBUF_G - 1) // NBUF_G

        def body(j, carry):
            for b in range(NBUF_G):
                ch = j * NBUF_G + b

                @pl.when(ch < n_ch)
                def _():
                    wait_gather(b)
                    start_write(ch, b)
                    wait_write(b)

                    @pl.when(ch + NBUF_G < n_ch)
                    def _():
                        start_gather(ch + NBUF_G, b)

            return carry

        lax.fori_loop(0, n_out, body, 0)

    return k(ts, td, srcp, dstp)


def _k2_body(gs_ref, gd_ref, at_ref, vals_ref, sel_ref, sw32_ref, swref_ref,
             swrbf_ref, whs_ref, w2b_ref, b2b_ref, msvle_ref, r1e_ref,
             r2e_ref, tuf1_ref, tuf2_ref, tm7_ref, tsum_ref, mvs_ref,
             outa_ref, outb_ref):
    f32 = jnp.float32
    bf16 = jnp.bfloat16
    Gsb = gs_ref[...]
    Gs = Gsb.astype(f32)
    Gd = gd_ref[...].astype(f32)
    at = at_ref[...]
    h0s = Gs[:, 0:128]
    h0s_b = Gsb[:, 0:128]
    H1O_b = Gsb[:, 128:176]
    cs = Gs[:, 264:267]
    cd = Gd[:, 128:131]
    diff = cd - cs
    r2 = jnp.sum(diff * diff, axis=1, keepdims=True)
    dist = jnp.sqrt(r2)
    u = diff / (dist + 1e-9)

    centers = lax.broadcasted_iota(jnp.int32, (BE, NRBF), 1).astype(f32) * (5.0 / (NRBF - 1))
    width = 5.0 / NRBF
    rbf = jnp.exp(-(((dist - centers) / width) ** 2))

    spread = jnp.dot(at, sel_ref[...], preferred_element_type=f32)
    oh32 = (spread == vals_ref[...]).astype(f32)
    refd = at[:, 5:6]
    dd = dist - refd
    has = (refd > 0).astype(f32)
    eref3 = jnp.concatenate([jnp.abs(dd), dd, has], axis=1)

    hid_in = (jnp.dot(oh32, sw32_ref[...], preferred_element_type=f32)
              + jnp.dot(eref3, swref_ref[...], preferred_element_type=f32)
              + jnp.dot(rbf, swrbf_ref[...], preferred_element_type=f32)
              + jnp.dot(h0s_b, whs_ref[...], preferred_element_type=f32)
              + Gd[:, 0:128])
    hid = hid_in * (1.0 / (1.0 + jnp.exp(-hid_in)))
    w = jnp.dot(hid.astype(bf16), w2b_ref[...],
                preferred_element_type=f32) + b2b_ref[...]

    H1O = Gs[:, 128:176]
    HL = Gs[:, 128:264]
    urot = jnp.concatenate([u[:, 1:3], u[:, 0:1]], axis=1)
    m7 = jnp.concatenate([u * u, u * urot, jnp.ones((BE, 1), f32)], axis=1)
    E1 = (jnp.dot(h0s_b, msvle_ref[...], preferred_element_type=f32)
          + jnp.dot(H1O_b, r1e_ref[...], preferred_element_type=f32))
    E2 = jnp.dot(H1O_b, r2e_ref[...], preferred_element_type=f32)
    F1 = (jnp.dot(u, tuf1_ref[...], preferred_element_type=f32)
          + jnp.dot(m7, tm7_ref[...], preferred_element_type=f32))
    F2 = jnp.dot(u, tuf2_ref[...], preferred_element_type=f32)
    Q = E1 * F1 - E2 * F2
    A1 = w[:, 128:264]
    A2 = w[:, 264:400]
    msgL = A1 * HL + A2 * Q

    D = H1O * F1[:, 0:48]
    dot1 = jnp.dot(D, tsum_ref[...], preferred_element_type=f32)
    wc = w[:, 400:416]
    msg0 = w[:, 0:128] * h0s + jnp.dot(wc * dot1, mvs_ref[...],
                                       preferred_element_type=f32)

    q = 0.25
    outa_ref[:, 0:128] = q * msg0
    outa_ref[:, 128:144] = jnp.zeros((BE, 16), f32)
    outb_ref[:, 0:136] = q * msgL
    outb_ref[:, 136:144] = jnp.zeros((BE, 8), f32)


def _edge_compute(gs, gd, attr, vals, sel, sw32, swref, swrbf, w_hs,
                  w2b, b2b, msvle, r1e, r2e, tuf1, tuf2, tm7, tsum, m_vs):
    grid = EPAD // BE
    const = lambda shape: pl.BlockSpec(shape, lambda i: (0,) * len(shape))
    return pl.pallas_call(
        _k2_body,
        grid=(grid,),
        in_specs=[
            pl.BlockSpec((BE, WS), lambda i: (i, 0)),
            pl.BlockSpec((BE, WD), lambda i: (i, 0)),
            pl.BlockSpec((BE, 8), lambda i: (i, 0)),
            const((1, 32)), const((8, 32)), const((32, 128)),
            const((3, 128)), const((16, 128)), const((128, 128)),
            const((128, 416)), const((1, 416)), const((128, 136)),
            const((48, 136)), const((48, 136)), const((3, 136)),
            const((3, 136)), const((7, 136)), const((48, 16)),
            const((16, 128)),
        ],
        out_specs=[
            pl.BlockSpec((BE, HALF), lambda i: (i, 0)),
            pl.BlockSpec((BE, HALF), lambda i: (i, 0)),
        ],
        out_shape=[
            jax.ShapeDtypeStruct((EPAD, HALF), jnp.float32),
            jax.ShapeDtypeStruct((EPAD, HALF), jnp.float32),
        ],
        interpret=False,
    )(gs, gd, attr, vals, sel, sw32, swref, swrbf, w_hs, w2b, b2b,
      msvle, r1e, r2e, tuf1, tuf2, tm7, tsum, m_vs)


def _sc_scatter(msga, msgb, dstp, zrows):
    mesh = plsc.VectorSubcoreMesh(core_axis_name="c", subcore_axis_name="s",
                                  num_cores=2, num_subcores=16)
    per_t = EPAD // 16
    n_ch = per_t // CH
    zch = SPROWS // 16
    out_ch = NPAD // CH // 16

    iblk = 8

    @functools.partial(
        pl.kernel,
        out_type=(jax.ShapeDtypeStruct((NPAD, HALF), jnp.float32),
                  jax.ShapeDtypeStruct((NPAD, HALF), jnp.float32)),
        mesh=mesh,
        compiler_params=pltpu.CompilerParams(use_tc_tiling_on_sc=False),
        scratch_types=(
            [pltpu.VMEM((iblk, CH), jnp.int32)]
            + [pltpu.VMEM((CH, HALF), jnp.float32) for _ in range(NBUF_S)]
            + [pltpu.VMEM_SHARED((SPROWS, HALF), jnp.float32)]
            + [pltpu.SemaphoreType.DMA for _ in range(NBUF_S)]
        ),
    )
    def k(ma_hbm, mb_hbm, dst_hbm, z_hbm, outa_hbm, outb_hbm,
          idx_v, *bufs):
        rows = bufs[0:NBUF_S]
        acc_sh = bufs[NBUF_S]
        rsem = bufs[NBUF_S + 1:2 * NBUF_S + 1]
        c = lax.axis_index("c")
        t = lax.axis_index("s")
        pltpu.sync_copy(z_hbm, acc_sh.at[pl.ds(t * zch, zch)])
        plsc.subcore_barrier()

        base = t * per_t

        def start_read(ch, b):
            off = base + ch * CH

            @pl.when(c == 0)
            def _():
                pltpu.async_copy(ma_hbm.at[pl.ds(off, CH)], rows[b], rsem[b])

            @pl.when(c == 1)
            def _():
                pltpu.async_copy(mb_hbm.at[pl.ds(off, CH)], rows[b], rsem[b])

        def wait_read(b):
            pltpu.make_async_copy(ma_hbm.at[pl.ds(0, CH)], rows[b], rsem[b]).wait()

        pltpu.sync_copy(dst_hbm.at[t, pl.ds(0, iblk)], idx_v)
        for b in range(NBUF_S):
            start_read(b, b)

        n_it = n_ch // NBUF_S
        per_blk = iblk // NBUF_S

        def body(j, carry):
            for b in range(NBUF_S):
                ch = j * NBUF_S + b
                row = (j % per_blk) * NBUF_S + b
                wait_read(b)
                pltpu.sync_copy(rows[b], acc_sh.at[idx_v.at[row]], add=True)

                @pl.when(ch + NBUF_S < n_ch)
                def _():
                    start_read(ch + NBUF_S, b)

            nxt = (j + 1) * NBUF_S

            @pl.when((nxt % iblk == 0) & (nxt < n_ch))
            def _():
                pltpu.sync_copy(dst_hbm.at[t, pl.ds(nxt, iblk)], idx_v)

            return carry

        lax.fori_loop(0, n_it, body, 0)
        plsc.subcore_barrier()

        def wbody(i, carry):
            cidx = t + i * 16

            @pl.when(c == 0)
            def _():
                pltpu.sync_copy(acc_sh.at[pl.ds(cidx * CH, CH)],
                                outa_hbm.at[pl.ds(cidx * CH, CH)])

            @pl.when(c == 1)
            def _():
                pltpu.sync_copy(acc_sh.at[pl.ds(cidx * CH, CH)],
                                outb_hbm.at[pl.ds(cidx * CH, CH)])

            return carry

        lax.fori_loop(0, out_ch, wbody, 0)

    return k(msga, msgb, dstp, zrows)


def _k4_body(aa_ref, ab_ref, hk_ref, te_ref, pa_ref, pb_ref, aw_ref, ab2_ref,
             out_ref):
    A = aa_ref[...]
    Bm = ab_ref[...]
    H = hk_ref[...]
    T = te_ref[...]
    U = (jnp.dot(A, pa_ref[...], preferred_element_type=jnp.float32)
         + jnp.dot(Bm, pb_ref[...], preferred_element_type=jnp.float32))
    u0 = U[:, 0:128]
    u0 = u0 * (1.0 / (1.0 + jnp.exp(-u0)))
    hn0 = H[:, 0:128] + u0
    hn1 = H[:, 128:264] + U[:, 128:264]

    p = jnp.dot(T, aw_ref[...], preferred_element_type=jnp.float32) + ab2_ref[...]
    scale = p[:, 0:128]
    shift = p[:, 128:256]
    g1o = p[:, 256:272]
    g1e = p[:, 272:288]
    g2 = p[:, 288:296]

    mu = jnp.mean(hn0, axis=1, keepdims=True)
    xc = hn0 - mu
    var = jnp.mean(xc * xc, axis=1, keepdims=True)
    o0 = xc * lax.rsqrt(var + 1e-5) * (1.0 + scale) + shift

    b1o = hn1[:, 0:48]
    b1e = hn1[:, 48:96]
    b2_ = hn1[:, 96:136]
    n1o = jnp.sqrt(jnp.sum(b1o * b1o, axis=1, keepdims=True) / 16.0 + 1e-5)
    n1e = jnp.sqrt(jnp.sum(b1e * b1e, axis=1, keepdims=True) / 16.0 + 1e-5)
    n2 = jnp.sqrt(jnp.sum(b2_ * b2_, axis=1, keepdims=True) / 8.0 + 1e-5)
    g1o3 = jnp.concatenate([1.0 + g1o] * 3, axis=1)
    g1e3 = jnp.concatenate([1.0 + g1e] * 3, axis=1)
    g25 = jnp.concatenate([1.0 + g2] * 5, axis=1)

    out_ref[:, 0:128] = o0
    out_ref[:, 128:176] = b1o / n1o * g1o3
    out_ref[:, 176:224] = b1e / n1e * g1e3
    out_ref[:, 224:264] = b2_ / n2 * g25


def _node_update(agga, aggb, hk, te, pa, pb, aw, ab2):
    grid = NPAD // BN
    return pl.pallas_call(
        _k4_body,
        grid=(grid,),
        in_specs=[
            pl.BlockSpec((BN, HALF), lambda i: (i, 0)),
            pl.BlockSpec((BN, HALF), lambda i: (i, 0)),
            pl.BlockSpec((BN, 264), lambda i: (i, 0)),
            pl.BlockSpec((BN, 128), lambda i: (i, 0)),
            pl.BlockSpec((HALF, 264), lambda i: (0, 0)),
            pl.BlockSpec((HALF, 264), lambda i: (0, 0)),
            pl.BlockSpec((128, 296), lambda i: (0, 0)),
            pl.BlockSpec((1, 296), lambda i: (0, 0)),
        ],
        out_specs=pl.BlockSpec((BN, 264), lambda i: (i, 0)),
        out_shape=jax.ShapeDtypeStruct((NPAD, 264), jnp.float32),
        interpret=False,
    )(agga, aggb, hk, te, pa, pb, aw, ab2)


def kernel(h, coords, edge_index, edge_type, edge_bond_type,
           edge_bond_conjugated, edge_bond_in_ring, edge_bond_stereo,
           edge_ref_dist, t_emb, E_type, E_bt, E_conj, E_ring, E_st,
           ref_W, ref_b, mlp_W1, mlp_b1, mlp_W2, mlp_b2,
           M_sv, M_vs, M_sl, P0, P1o, P1e, P2, A_W, A_b):
    f32 = jnp.float32
    bf16 = jnp.bfloat16
    n = h.shape[0]
    e = edge_index.shape[1]

    h_k = h[:, _PERM]
    h_kp = jnp.pad(h_k, ((0, NPAD - n), (0, 0)))
    co8 = jnp.pad(coords.astype(f32), ((0, NPAD - n), (0, 5)))
    te_p = jnp.pad(t_emb, ((0, NPAD - n), (0, 0)))
    tsrc = jnp.concatenate(
        [h_kp.astype(bf16), co8.astype(bf16),
         jnp.zeros((NPAD, WS - 272), bf16)], axis=1)

    src = edge_index[0].astype(jnp.int32)
    dst = edge_index[1].astype(jnp.int32)
    per_w = EPAD // NW_SC
    srcp = jnp.pad(src, (0, EPAD - e)).reshape(NW_SC, per_w // CH, CH)
    dst_g = jnp.pad(dst, (0, EPAD - e)).reshape(NW_SC, per_w // CH, CH)
    per_t = EPAD // 16
    dst_s = jnp.pad(dst, (0, EPAD - e),
                    constant_values=DUMP).reshape(16, per_t // CH, CH)

    attr = jnp.concatenate([
        edge_type[:, None].astype(f32),
        edge_bond_type[:, None].astype(f32),
        edge_bond_conjugated[:, None].astype(f32),
        edge_bond_in_ring[:, None].astype(f32),
        edge_bond_stereo[:, None].astype(f32),
        edge_ref_dist[:, None].astype(f32),
        jnp.zeros((e, 2), f32),
    ], axis=1)
    attr = jnp.pad(attr, ((0, EPAD - e), (0, 0)))

    w1_rbf = mlp_W1[0:16]
    w1_et = mlp_W1[16:32]
    w1_bt = mlp_W1[32:40]
    w1_cj = mlp_W1[40:44]
    w1_rg = mlp_W1[44:48]
    w1_st = mlp_W1[48:52]
    w1_rf = mlp_W1[52:60]
    w1_hs = mlp_W1[60:188]
    w1_hd = mlp_W1[188:316]
    w1_te = mlp_W1[316:444]
    sw32 = jnp.concatenate([
        E_type @ w1_et, E_bt @ w1_bt, E_conj @ w1_cj, E_ring @ w1_rg,
        E_st @ w1_st, jnp.zeros((6, 128), f32),
    ], axis=0)
    swref = ref_W @ w1_rf
    b1f = (mlp_b1 + ref_b @ w1_rf)[None, :]

    seln = np.zeros((8, 32), np.float32)
    valn = np.zeros((1, 32), np.float32)
    _offs = [(0, 0, 9), (1, 9, 6), (2, 15, 3), (3, 18, 3), (4, 21, 5)]
    for col, off, kk in _offs:
        for j in range(kk):
            seln[col, off + j] = 1.0
            valn[0, off + j] = float(j)
    for j in range(26, 32):
        seln[6, j] = 1.0
        valn[0, j] = -1.0
    sel = jnp.asarray(seln)
    vals = jnp.asarray(valn)

    expn = np.zeros((224, 416), np.float32)
    for j in range(128):
        expn[j, j] = 1.0
    for k in range(3):
        for v in range(16):
            expn[144 + v, 128 + 16 * k + v] = 1.0
            expn[176 + v, 176 + 16 * k + v] = 1.0
            expn[128 + v, 264 + 16 * k + v] = 1.0
            expn[192 + v, 312 + 16 * k + v] = 1.0
    for k in range(5):
        for l in range(8):
            expn[208 + l, 224 + 8 * k + l] = 1.0
            expn[216 + l, 360 + 8 * k + l] = 1.0
    for v in range(16):
        expn[160 + v, 400 + v] = 1.0
    expj = jnp.asarray(expn)
    w2b = mlp_W2 @ expj
    b2b = (mlp_b2 @ expj)[None, :]

    tile_sv = np.zeros((16, 48), np.float32)
    for k in range(3):
        for v in range(16):
            tile_sv[v, 16 * k + v] = 1.0
    tile_sl = np.zeros((8, 40), np.float32)
    for k in range(5):
        for l in range(8):
            tile_sl[l, 8 * k + l] = 1.0
    msvle = jnp.concatenate([
        M_sv @ jnp.asarray(tile_sv), jnp.zeros((128, 48), f32),
        M_sl @ jnp.asarray(tile_sl)], axis=1)

    r1n = np.zeros((48, 136), np.float32)
    r2n = np.zeros((48, 136), np.float32)
    for k in range(3):
        for v in range(16):
            r1n[16 * ((k + 1) % 3) + v, 48 + 16 * k + v] = 1.0
            r2n[16 * ((k + 2) % 3) + v, 48 + 16 * k + v] = 1.0
    r1e = jnp.asarray(r1n)
    r2e = jnp.asarray(r2n)

    t1n = np.zeros((3, 136), np.float32)
    t2n = np.zeros((3, 136), np.float32)
    for k in range(3):
        for v in range(16):
            t1n[k, 16 * k + v] = _SQRT3
            t1n[(k + 2) % 3, 48 + 16 * k + v] = _SQRT3
            t2n[(k + 1) % 3, 48 + 16 * k + v] = _SQRT3
    tuf1 = jnp.asarray(t1n)
    tuf2 = jnp.asarray(t2n)

    tmn = np.zeros((7, 136), np.float32)
    y2rows = [
        {3: _C15},
        {4: _C15},
        {2: 3.0 * _C5H, 6: -_C5H},
        {5: _C15},
        {0: _C15 / 2.0, 1: -_C15 / 2.0},
    ]
    for k in range(5):
        for row, coef in y2rows[k].items():
            for l in range(8):
                tmn[row, 96 + 8 * k + l] = coef
    tm7 = jnp.asarray(tmn)

    tsn = np.zeros((48, 16), np.float32)
    for k in range(3):
        for v in range(16):
            tsn[16 * k + v, v] = 1.0
    tsum = jnp.asarray(tsn)

    pbig = jax.scipy.linalg.block_diag(
        P0, P1o, P1o, P1o, P1e, P1e, P1e, P2, P2, P2, P2, P2)
    pa = jnp.pad(pbig[0:128], ((0, HALF - 128), (0, 0)))
    pb = jnp.pad(pbig[128:264], ((0, HALF - 136), (0, 0)))
    ab2 = A_b[None, :]
    zrows = jnp.zeros((SPROWS // 16, HALF), f32)

    tdst = _node_pre(h_kp[:, 0:128], te_p, co8, w1_hd, w1_te, b1f)
    gs, gd = _sc_gather(tsrc, tdst, srcp, dst_g)
    msga, msgb = _edge_compute(gs, gd, attr, vals, sel, sw32, swref, w1_rbf,
                               w1_hs.astype(bf16), w2b.astype(bf16), b2b,
                               msvle.astype(bf16), r1e.astype(bf16),
                               r2e.astype(bf16), tuf1, tuf2,
                               tm7, tsum, M_vs)
    agga, aggb = _sc_scatter(msga, msgb, dst_s, zrows)
    out_k = _node_update(agga, aggb, h_kp, te_p, pa, pb, A_W, ab2)
    return out_k[:n][:, _INVPERM]
